# baseline (speedup 1.0000x reference)
import functools

import jax
import jax.numpy as jnp
from jax import lax
from jax.experimental import pallas as pl
from jax.experimental.pallas import tpu as pltpu
from jax.experimental.pallas import tpu_sc as plsc

F32 = jnp.float32
I32 = jnp.int32

_N = 5000
_N2 = 2 * _N
_E = 320000
_EPAD = 327680
_C = 128
_EPT = _EPAD // 16
_NITER = _EPT // _C
_HEPT = _EPAD // 32
_HNITER = _HEPT // _C
_SR = 64
_SNIT = _HEPT // _SR
_DUMP = 16
_NACC = _N + _DUMP
_P = 10000
_NEG = 50000
_Q = _P + _NEG
_QPAD = 61440
_QPW = _QPAD // 32

_mesh = plsc.VectorSubcoreMesh(core_axis_name="c", subcore_axis_name="s",
                               num_cores=2, num_subcores=16)


def _zero_fill(z_v, nrow, ncol):
  def row(r, carry):
    for j in range(ncol // 16):
      z_v[r, pl.ds(j * 16, 16)] = jnp.zeros((16,), F32)
    return carry
  lax.fori_loop(0, nrow, row, 0)


def _stats_body(e1_hbm, isrc_hbm, idst_hbm, out_hbm,
                accs_sp, accd_sp, u0, u1,
                xs0, xs1, xd0, xd1, ss0, ss1, sd0, sd1):
  c = lax.axis_index("c")
  s = lax.axis_index("s")
  upd = (u0, u1)
  ixs = (xs0, xs1)
  ixd = (xd0, xd1)
  sss = (ss0, ss1)
  ssd = (sd0, sd1)
  _zero_fill(u0, _SR, 128)
  for o in (0, 64, 128, 192, 256):
    zoff = jnp.minimum(s * 320 + o, _N - _SR)
    pltpu.sync_copy(u0, accs_sp.at[pl.ds(zoff, _SR)])
    pltpu.sync_copy(u0, accd_sp.at[pl.ds(zoff, _SR)])
  plsc.subcore_barrier()

  def load(i, b):
    base = c * (_EPAD // 2) + s * _HEPT + i * _SR
    pltpu.sync_copy(isrc_hbm.at[pl.ds(base, _SR)], ixs[b])
    pltpu.sync_copy(idst_hbm.at[pl.ds(base, _SR)], ixd[b])
    pltpu.sync_copy(e1_hbm.at[pl.ds(base, _SR)], upd[b])

  def s_start(b):
    pltpu.async_copy(upd[b], accs_sp.at[ixs[b]], sss[b], add=True)
    pltpu.async_copy(upd[b], accd_sp.at[ixd[b]], ssd[b], add=True)

  def s_wait(b):
    pltpu.make_async_copy(upd[b], accs_sp.at[ixs[b]], sss[b]).wait()
    pltpu.make_async_copy(upd[b], accd_sp.at[ixd[b]], ssd[b]).wait()

  load(0, 0)
  s_start(0)
  load(1, 1)
  s_start(1)

  def step(j, carry):
    for b in range(2):
      i = 2 * j + b
      s_wait(b)
      load(i, b)
      s_start(b)
    return carry

  lax.fori_loop(1, _SNIT // 2, step, 0)
  s_wait(0)
  s_wait(1)
  plsc.subcore_barrier()
  ooff = jnp.minimum(s * 320, _N - 320)
  pltpu.sync_copy(accs_sp.at[pl.ds(ooff, 320)],
                  out_hbm.at[c, 0, pl.ds(ooff, 320)])
  pltpu.sync_copy(accd_sp.at[pl.ds(ooff, 320)],
                  out_hbm.at[c, 1, pl.ds(ooff, 320)])


_stats_call = functools.partial(
    pl.kernel,
    _stats_body,
    out_type=jax.ShapeDtypeStruct((2, 2, _N, 128), F32),
    mesh=_mesh,
    scratch_types=[
        pltpu.VMEM_SHARED((_N, 128), F32),
        pltpu.VMEM_SHARED((_N, 128), F32),
        pltpu.VMEM((_SR, 128), F32),
        pltpu.VMEM((_SR, 128), F32),
        pltpu.VMEM((_SR,), I32),
        pltpu.VMEM((_SR,), I32),
        pltpu.VMEM((_SR,), I32),
        pltpu.VMEM((_SR,), I32),
        pltpu.SemaphoreType.DMA,
        pltpu.SemaphoreType.DMA,
        pltpu.SemaphoreType.DMA,
        pltpu.SemaphoreType.DMA,
    ],
)()


def _spmm_body(tbl_hbm, gs_hbm, out_hbm,
               acc_sp, b0, b1, b2, b3, i0, i1, i2, i3,
               g0, g1, g2, g3, t0, t1, t2, t3):
  c = lax.axis_index("c")
  s = lax.axis_index("s")
  bufs = (b0, b1, b2, b3)
  ibufs = (i0, i1, i2, i3)
  gsem = (g0, g1, g2, g3)
  ssem = (t0, t1, t2, t3)
  _zero_fill(b0, _C, 128)
  for o in (0, 128, 192):
    zoff = jnp.minimum(s * 320 + o, _NACC - _C)
    pltpu.sync_copy(b0, acc_sp.at[pl.ds(zoff, _C)])
  plsc.subcore_barrier()

  def load_idx(i, b):
    pltpu.sync_copy(gs_hbm.at[c, s * _NITER + i], ibufs[b])

  def g_start(b):
    pltpu.async_copy(tbl_hbm.at[ibufs[b].at[0]], bufs[b], gsem[b])

  def g_wait(b):
    pltpu.make_async_copy(tbl_hbm.at[ibufs[b].at[0]], bufs[b], gsem[b]).wait()

  def s_start(b):
    pltpu.async_copy(bufs[b], acc_sp.at[ibufs[b].at[1]], ssem[b], add=True)

  def s_wait(b):
    pltpu.make_async_copy(bufs[b], acc_sp.at[ibufs[b].at[1]], ssem[b]).wait()

  load_idx(0, 0)
  g_start(0)
  load_idx(1, 1)
  g_start(1)
  load_idx(2, 2)
  g_start(2)
  g_wait(0)
  s_start(0)
  load_idx(3, 3)
  g_start(3)
  g_wait(1)
  s_start(1)

  def step(j, carry):
    for b in range(4):
      i = 4 * j + b
      s_wait(b)
      load_idx(i, b)
      g_start(b)
      bp = (b + 2) % 4
      g_wait(bp)
      s_start(bp)
    return carry

  lax.fori_loop(1, _NITER // 4, step, 0)
  g_wait(2)
  s_start(2)
  g_wait(3)
  s_start(3)
  for b in range(4):
    s_wait(b)
  plsc.subcore_barrier()
  ooff = jnp.minimum(s * 320, _NACC - 320)
  pltpu.sync_copy(acc_sp.at[pl.ds(ooff, 320)], out_hbm.at[c, pl.ds(ooff, 320)])


_spmm_call = functools.partial(
    pl.kernel,
    _spmm_body,
    out_type=jax.ShapeDtypeStruct((2, _NACC, 128), F32),
    mesh=_mesh,
    scratch_types=[
        pltpu.VMEM_SHARED((_NACC, 128), F32),
        pltpu.VMEM((_C, 128), F32),
        pltpu.VMEM((_C, 128), F32),
        pltpu.VMEM((_C, 128), F32),
        pltpu.VMEM((_C, 128), F32),
        pltpu.VMEM((2, _C), I32),
        pltpu.VMEM((2, _C), I32),
        pltpu.VMEM((2, _C), I32),
        pltpu.VMEM((2, _C), I32),
        pltpu.SemaphoreType.DMA,
        pltpu.SemaphoreType.DMA,
        pltpu.SemaphoreType.DMA,
        pltpu.SemaphoreType.DMA,
        pltpu.SemaphoreType.DMA,
        pltpu.SemaphoreType.DMA,
        pltpu.SemaphoreType.DMA,
        pltpu.SemaphoreType.DMA,
    ],
)()


def _pairs_body(tbl_hbm, qa_hbm, qb_hbm, oa_hbm, ob_hbm,
                ia_v, ib_v, ra_v, rb_v):
  c = lax.axis_index("c")
  s = lax.axis_index("s")
  wid = c * 16 + s

  def step(i, carry):
    base = wid * _QPW + i * _C
    pltpu.sync_copy(qa_hbm.at[pl.ds(base, _C)], ia_v)
    pltpu.sync_copy(qb_hbm.at[pl.ds(base, _C)], ib_v)
    pltpu.sync_copy(tbl_hbm.at[ia_v], ra_v)
    pltpu.sync_copy(tbl_hbm.at[ib_v], rb_v)
    pltpu.sync_copy(ra_v, oa_hbm.at[pl.ds(base, _C)])
    pltpu.sync_copy(rb_v, ob_hbm.at[pl.ds(base, _C)])
    return carry

  lax.fori_loop(0, _QPW // _C, step, 0)


_pairs_call = functools.partial(
    pl.kernel,
    _pairs_body,
    out_type=(jax.ShapeDtypeStruct((_QPAD, 128), F32),
              jax.ShapeDtypeStruct((_QPAD, 128), F32)),
    mesh=_mesh,
    scratch_types=[
        pltpu.VMEM((_C,), I32),
        pltpu.VMEM((_C,), I32),
        pltpu.VMEM((_C, 128), F32),
        pltpu.VMEM((_C, 128), F32),
    ],
)()


def _full(shape):
  return pl.BlockSpec(shape, lambda i: (0,) * len(shape))


def _tc1_body(h_ref, wue, wie, wna, wnc, wsc, wsa, x1_ref, s1_ref):
  cust = pl.program_id(0) < 5
  we = jnp.where(cust, wue[...], wie[...])
  hb = jnp.dot(h_ref[...], we, preferred_element_type=F32)
  wn = jnp.where(cust, wna[...], wnc[...])
  x1_ref[...] = jnp.dot(hb, wn, preferred_element_type=F32)
  ws = jnp.where(cust, wsc[...], wsa[...])
  s1_ref[...] = jnp.dot(hb, ws, preferred_element_type=F32)


def _tc1(h, wue, wie, wna, wnc, wsc, wsa):
  blk = pl.BlockSpec((1000, 128), lambda i: (i, 0))
  return pl.pallas_call(
      _tc1_body,
      grid=(10,),
      in_specs=[blk] + [_full((128, 128))] * 6,
      out_specs=[blk, blk],
      out_shape=[jax.ShapeDtypeStruct((_N2, 128), F32)] * 2,
  )(h, wue, wie, wna, wnc, wsc, wsa)


def _tc2_body(s1_ref, y1_ref, st_ref, wnbc, wnba, wpa, wpc, wsc2, wsa2,
              x2_ref, s2_ref):
  cust = pl.program_id(0) < 5
  st = st_ref[...]
  deg = jnp.maximum(st[:, 4], 1.0)
  wnb = jnp.where(cust, wnbc[...], wnba[...])
  agg = (y1_ref[...] + jnp.dot(st, wnb, preferred_element_type=F32)) / deg[:, None]
  h1 = jnp.maximum(s1_ref[...] + agg, 0.0)
  x2_ref[...] = jnp.dot(h1, jnp.where(cust, wpa[...], wpc[...]),
                        preferred_element_type=F32)
  s2_ref[...] = jnp.dot(h1, jnp.where(cust, wsc2[...], wsa2[...]),
                        preferred_element_type=F32)


def _tc2(s1, y1, st, wnbc, wnba, wpa, wpc, wsc2, wsa2):
  blk128 = pl.BlockSpec((1000, 128), lambda i: (i, 0))
  blk64 = pl.BlockSpec((1000, 64), lambda i: (i, 0))
  blk16 = pl.BlockSpec((1000, 16), lambda i: (i, 0))
  return pl.pallas_call(
      _tc2_body,
      grid=(10,),
      in_specs=[blk128, blk128, blk16,
                _full((16, 128)), _full((16, 128)),
                _full((128, 128)), _full((128, 128)),
                _full((128, 64)), _full((128, 64))],
      out_specs=[blk128, blk64],
      out_shape=[jax.ShapeDtypeStruct((_N2, 128), F32),
                 jax.ShapeDtypeStruct((_N2, 64), F32)],
  )(s1, y1, st, wnbc, wnba, wpa, wpc, wsc2, wsa2)


def _tc3_body(s2_ref, y2_ref, st_ref, wnbc, wnba, h2_ref):
  cust = pl.program_id(0) < 5
  st = st_ref[...]
  deg = jnp.maximum(st[:, 4], 1.0)
  wnb = jnp.where(cust, wnbc[...], wnba[...])
  agg = (y2_ref[:, :64] + jnp.dot(st, wnb, preferred_element_type=F32)) / deg[:, None]
  h2 = s2_ref[...] + agg
  h2_ref[...] = jnp.concatenate([h2, jnp.zeros_like(h2)], axis=1)


def _tc3(s2, y2, st, wnbc, wnba):
  blk128 = pl.BlockSpec((1000, 128), lambda i: (i, 0))
  blk64 = pl.BlockSpec((1000, 64), lambda i: (i, 0))
  blk16 = pl.BlockSpec((1000, 16), lambda i: (i, 0))
  return pl.pallas_call(
      _tc3_body,
      grid=(10,),
      in_specs=[blk64, blk128, blk16, _full((16, 64)), _full((16, 64))],
      out_specs=blk128,
      out_shape=jax.ShapeDtypeStruct((_N2, 128), F32),
  )(s2, y2, st, wnbc, wnba)


def _tc4_body(ga_ref, gb_ref, out_ref):
  a = ga_ref[...]
  b = gb_ref[...]
  dot = jnp.sum(a * b, axis=1)
  na = jnp.sqrt(jnp.sum(a * a, axis=1))
  nb = jnp.sqrt(jnp.sum(b * b, axis=1))
  out_ref[...] = dot / (na * nb + 1e-8)


def _tc4(ga, gb):
  blk = pl.BlockSpec((6144, 128), lambda i: (i, 0))
  return pl.pallas_call(
      _tc4_body,
      grid=(10,),
      in_specs=[blk, blk],
      out_specs=pl.BlockSpec((6144,), lambda i: (i,)),
      out_shape=jax.ShapeDtypeStruct((_QPAD,), F32),
  )(ga, gb)


def _pad16(w4):
  return jnp.concatenate([w4, jnp.zeros((12, w4.shape[1]), F32)], axis=0)


def kernel(h_customer, h_article, e_feat, src_idx, dst_idx,
           pos_src, pos_dst, neg_src, neg_dst,
           W_ue, W_ie, W1a_n, W1a_s, W1c_n, W1c_s,
           W2a_n, W2a_s, W2c_n, W2c_s):
  src_idx = src_idx.astype(I32)
  dst_idx = dst_idx.astype(I32)
  npad = _EPAD - _E
  padv = jnp.arange(npad, dtype=I32) % _N
  dumpv = _N + (jnp.arange(npad, dtype=I32) % _DUMP)
  src_g = jnp.concatenate([src_idx, padv])
  dst_g = jnp.concatenate([dst_idx, padv])
  gidx = jnp.stack([dst_g + _N, src_g]).reshape(2, _EPAD // _C, _C)
  sidx = jnp.stack([jnp.concatenate([src_idx, dumpv]),
                    jnp.concatenate([dst_idx, dumpv])
                    ]).reshape(2, _EPAD // _C, _C)
  gs = jnp.stack([gidx, sidx], axis=2)
  e1 = jnp.concatenate(
      [e_feat, jnp.ones((_E, 1), F32), jnp.zeros((_E, 123), F32)], axis=1)
  e1 = jnp.concatenate([e1, jnp.zeros((npad, 128), F32)], axis=0)

  qpadv = jnp.arange(_QPAD - _Q, dtype=I32) % _N
  qa = jnp.concatenate([pos_src.astype(I32), neg_src.astype(I32), qpadv])
  qb = jnp.concatenate([pos_dst.astype(I32), neg_dst.astype(I32), qpadv]) + _N

  H = jnp.concatenate([h_customer, h_article], axis=0)

  sp = _stats_call(e1, src_g, dst_g)[:, :, :, :16]
  stats = jnp.concatenate([sp[0, 0] + sp[1, 0], sp[0, 1] + sp[1, 1]])

  x1, s1 = _tc1(H, W_ue, W_ie, W1a_n[:128], W1c_n[:128], W1c_s, W1a_s)
  y1 = _spmm_call(x1, gs)[:, :_N, :].reshape(_N2, 128)
  wpa = jnp.concatenate([W2a_n[:128], jnp.zeros((128, 64), F32)], axis=1)
  wpc = jnp.concatenate([W2c_n[:128], jnp.zeros((128, 64), F32)], axis=1)
  x2, s2 = _tc2(s1, y1, stats, _pad16(W1c_n[128:]), _pad16(W1a_n[128:]),
                wpa, wpc, W2c_s, W2a_s)
  y2 = _spmm_call(x2, gs)[:, :_N, :].reshape(_N2, 128)
  h2 = _tc3(s2, y2, stats, _pad16(W2c_n[128:]), _pad16(W2a_n[128:]))

  ga, gb = _pairs_call(h2, qa, qb)
  scores = _tc4(ga, gb)

  return h2[:_N, :64], h2[_N:, :64], scores[:_P], scores[_P:_Q]


# speedup vs baseline: 1.2722x; 1.2722x over previous
import functools

import jax
import jax.numpy as jnp
from jax import lax
from jax.experimental import pallas as pl
from jax.experimental.pallas import tpu as pltpu
from jax.experimental.pallas import tpu_sc as plsc

F32 = jnp.float32
I32 = jnp.int32

_N = 5000
_N2 = 2 * _N
_E = 320000
_EPAD = 327680
_C = 128
_EPT = _EPAD // 16
_NITER = _EPT // _C
_HEPT = _EPAD // 32
_HNITER = _HEPT // _C
_SR = 64
_SNIT = _HEPT // _SR
_DUMP = 16
_NACC = _N + _DUMP
_P = 10000
_NEG = 50000
_Q = _P + _NEG
_QPAD = 61440
_QPW = _QPAD // 32

_mesh = plsc.VectorSubcoreMesh(core_axis_name="c", subcore_axis_name="s",
                               num_cores=2, num_subcores=16)


def _zero_fill(z_v, nrow, ncol):
  def row(r, carry):
    for j in range(ncol // 16):
      z_v[r, pl.ds(j * 16, 16)] = jnp.zeros((16,), F32)
    return carry
  lax.fori_loop(0, nrow, row, 0)


def _stats_body(e1_hbm, isrc_hbm, idst_hbm, out_hbm,
                accs_sp, accd_sp, u0, u1, e16_v, xs0, xs1, xd0, xd1,
                ss0, ss1, sd0, sd1):
  c = lax.axis_index("c")
  s = lax.axis_index("s")
  upd = (u0, u1)
  ixs = (xs0, xs1)
  ixd = (xd0, xd1)
  ssem = (ss0, ss1)
  dsem = (sd0, sd1)
  _zero_fill(u0, _C, 128)
  _zero_fill(u1, _C, 128)
  for o in (0, 128, 192):
    zoff = jnp.minimum(s * 320 + o, _N - _C)
    pltpu.sync_copy(u0, accs_sp.at[pl.ds(zoff, _C)])
    pltpu.sync_copy(u0, accd_sp.at[pl.ds(zoff, _C)])
  plsc.subcore_barrier()

  def load_expand(i, b):
    base = c * (_EPAD // 2) + s * _HEPT + i * _C
    pltpu.sync_copy(isrc_hbm.at[pl.ds(base, _C)], ixs[b])
    pltpu.sync_copy(idst_hbm.at[pl.ds(base, _C)], ixd[b])
    pltpu.sync_copy(e1_hbm.at[pl.ds(base, _C)], e16_v)

    def expand(r8, c2):
      for k in range(8):
        r = r8 * 8 + k
        upd[b][r, pl.ds(0, 16)] = e16_v[r, pl.ds(0, 16)]
      return c2

    lax.fori_loop(0, _C // 8, expand, 0)

  def s_start(b):
    pltpu.async_copy(upd[b], accs_sp.at[ixs[b]], ssem[b], add=True)
    pltpu.async_copy(upd[b], accd_sp.at[ixd[b]], dsem[b], add=True)

  def s_wait(b):
    pltpu.make_async_copy(upd[b], accs_sp.at[ixs[b]], ssem[b]).wait()
    pltpu.make_async_copy(upd[b], accd_sp.at[ixd[b]], dsem[b]).wait()

  load_expand(0, 0)
  s_start(0)
  load_expand(1, 1)
  s_start(1)

  def step(j, carry):
    for b in range(2):
      i = 2 * j + b
      s_wait(b)
      load_expand(i, b)
      s_start(b)
    return carry

  lax.fori_loop(1, _HNITER // 2, step, 0)
  s_wait(0)
  s_wait(1)
  plsc.subcore_barrier()
  ooff = jnp.minimum(s * 320, _N - 320)
  pltpu.sync_copy(accs_sp.at[pl.ds(ooff, 320)],
                  out_hbm.at[c, 0, pl.ds(ooff, 320)])
  pltpu.sync_copy(accd_sp.at[pl.ds(ooff, 320)],
                  out_hbm.at[c, 1, pl.ds(ooff, 320)])


_stats_call = functools.partial(
    pl.kernel,
    _stats_body,
    out_type=jax.ShapeDtypeStruct((2, 2, _N, 128), F32),
    mesh=_mesh,
    scratch_types=[
        pltpu.VMEM_SHARED((_N, 128), F32),
        pltpu.VMEM_SHARED((_N, 128), F32),
        pltpu.VMEM((_C, 128), F32),
        pltpu.VMEM((_C, 128), F32),
        pltpu.VMEM((_C, 16), F32),
        pltpu.VMEM((_C,), I32),
        pltpu.VMEM((_C,), I32),
        pltpu.VMEM((_C,), I32),
        pltpu.VMEM((_C,), I32),
        pltpu.SemaphoreType.DMA,
        pltpu.SemaphoreType.DMA,
        pltpu.SemaphoreType.DMA,
        pltpu.SemaphoreType.DMA,
    ],
)()


def _spmm_body(tbl_hbm, gs_hbm, out_hbm,
               acc_sp, b0, b1, b2, b3, i0, i1, i2, i3,
               g0, g1, g2, g3, t0, t1, t2, t3):
  c = lax.axis_index("c")
  s = lax.axis_index("s")
  bufs = (b0, b1, b2, b3)
  ibufs = (i0, i1, i2, i3)
  gsem = (g0, g1, g2, g3)
  ssem = (t0, t1, t2, t3)
  _zero_fill(b0, _C, 128)
  for o in (0, 128, 192):
    zoff = jnp.minimum(s * 320 + o, _NACC - _C)
    pltpu.sync_copy(b0, acc_sp.at[pl.ds(zoff, _C)])
  plsc.subcore_barrier()

  def load_idx(i, b):
    pltpu.sync_copy(gs_hbm.at[c, s * _NITER + i], ibufs[b])

  def g_start(b):
    pltpu.async_copy(tbl_hbm.at[ibufs[b].at[0]], bufs[b], gsem[b])

  def g_wait(b):
    pltpu.make_async_copy(tbl_hbm.at[ibufs[b].at[0]], bufs[b], gsem[b]).wait()

  def s_start(b):
    pltpu.async_copy(bufs[b], acc_sp.at[ibufs[b].at[1]], ssem[b], add=True)

  def s_wait(b):
    pltpu.make_async_copy(bufs[b], acc_sp.at[ibufs[b].at[1]], ssem[b]).wait()

  load_idx(0, 0)
  g_start(0)
  load_idx(1, 1)
  g_start(1)
  load_idx(2, 2)
  g_start(2)
  g_wait(0)
  s_start(0)
  load_idx(3, 3)
  g_start(3)
  g_wait(1)
  s_start(1)

  def step(j, carry):
    for b in range(4):
      i = 4 * j + b
      s_wait(b)
      load_idx(i, b)
      g_start(b)
      bp = (b + 2) % 4
      g_wait(bp)
      s_start(bp)
    return carry

  lax.fori_loop(1, _NITER // 4, step, 0)
  g_wait(2)
  s_start(2)
  g_wait(3)
  s_start(3)
  for b in range(4):
    s_wait(b)
  plsc.subcore_barrier()
  ooff = jnp.minimum(s * 320, _NACC - 320)
  pltpu.sync_copy(acc_sp.at[pl.ds(ooff, 320)], out_hbm.at[c, pl.ds(ooff, 320)])


_spmm_call = functools.partial(
    pl.kernel,
    _spmm_body,
    out_type=jax.ShapeDtypeStruct((2, _NACC, 128), F32),
    mesh=_mesh,
    scratch_types=[
        pltpu.VMEM_SHARED((_NACC, 128), F32),
        pltpu.VMEM((_C, 128), F32),
        pltpu.VMEM((_C, 128), F32),
        pltpu.VMEM((_C, 128), F32),
        pltpu.VMEM((_C, 128), F32),
        pltpu.VMEM((2, _C), I32),
        pltpu.VMEM((2, _C), I32),
        pltpu.VMEM((2, _C), I32),
        pltpu.VMEM((2, _C), I32),
        pltpu.SemaphoreType.DMA,
        pltpu.SemaphoreType.DMA,
        pltpu.SemaphoreType.DMA,
        pltpu.SemaphoreType.DMA,
        pltpu.SemaphoreType.DMA,
        pltpu.SemaphoreType.DMA,
        pltpu.SemaphoreType.DMA,
        pltpu.SemaphoreType.DMA,
    ],
)()


def _pairs_body(tbl_hbm, qa_hbm, qb_hbm, oa_hbm, ob_hbm,
                ia_v, ib_v, ra_v, rb_v):
  c = lax.axis_index("c")
  s = lax.axis_index("s")
  wid = c * 16 + s

  def step(i, carry):
    base = wid * _QPW + i * _C
    pltpu.sync_copy(qa_hbm.at[pl.ds(base, _C)], ia_v)
    pltpu.sync_copy(qb_hbm.at[pl.ds(base, _C)], ib_v)
    pltpu.sync_copy(tbl_hbm.at[ia_v], ra_v)
    pltpu.sync_copy(tbl_hbm.at[ib_v], rb_v)
    pltpu.sync_copy(ra_v, oa_hbm.at[pl.ds(base, _C)])
    pltpu.sync_copy(rb_v, ob_hbm.at[pl.ds(base, _C)])
    return carry

  lax.fori_loop(0, _QPW // _C, step, 0)


_pairs_call = functools.partial(
    pl.kernel,
    _pairs_body,
    out_type=(jax.ShapeDtypeStruct((_QPAD, 128), F32),
              jax.ShapeDtypeStruct((_QPAD, 128), F32)),
    mesh=_mesh,
    scratch_types=[
        pltpu.VMEM((_C,), I32),
        pltpu.VMEM((_C,), I32),
        pltpu.VMEM((_C, 128), F32),
        pltpu.VMEM((_C, 128), F32),
    ],
)()


def _full(shape):
  return pl.BlockSpec(shape, lambda i: (0,) * len(shape))


def _tc1_body(h_ref, wue, wie, wna, wnc, wsc, wsa, x1_ref, s1_ref):
  cust = pl.program_id(0) < 5
  we = jnp.where(cust, wue[...], wie[...])
  hb = jnp.dot(h_ref[...], we, preferred_element_type=F32)
  wn = jnp.where(cust, wna[...], wnc[...])
  x1_ref[...] = jnp.dot(hb, wn, preferred_element_type=F32)
  ws = jnp.where(cust, wsc[...], wsa[...])
  s1_ref[...] = jnp.dot(hb, ws, preferred_element_type=F32)


def _tc1(h, wue, wie, wna, wnc, wsc, wsa):
  blk = pl.BlockSpec((1000, 128), lambda i: (i, 0))
  return pl.pallas_call(
      _tc1_body,
      grid=(10,),
      in_specs=[blk] + [_full((128, 128))] * 6,
      out_specs=[blk, blk],
      out_shape=[jax.ShapeDtypeStruct((_N2, 128), F32)] * 2,
  )(h, wue, wie, wna, wnc, wsc, wsa)


def _tc2_body(s1_ref, y1_ref, st_ref, wnbc, wnba, wpa, wpc, wsc2, wsa2,
              x2_ref, s2_ref):
  cust = pl.program_id(0) < 5
  st = st_ref[...]
  deg = jnp.maximum(st[:, 4], 1.0)
  wnb = jnp.where(cust, wnbc[...], wnba[...])
  agg = (y1_ref[...] + jnp.dot(st, wnb, preferred_element_type=F32)) / deg[:, None]
  h1 = jnp.maximum(s1_ref[...] + agg, 0.0)
  x2_ref[...] = jnp.dot(h1, jnp.where(cust, wpa[...], wpc[...]),
                        preferred_element_type=F32)
  s2_ref[...] = jnp.dot(h1, jnp.where(cust, wsc2[...], wsa2[...]),
                        preferred_element_type=F32)


def _tc2(s1, y1, st, wnbc, wnba, wpa, wpc, wsc2, wsa2):
  blk128 = pl.BlockSpec((1000, 128), lambda i: (i, 0))
  blk64 = pl.BlockSpec((1000, 64), lambda i: (i, 0))
  blk16 = pl.BlockSpec((1000, 16), lambda i: (i, 0))
  return pl.pallas_call(
      _tc2_body,
      grid=(10,),
      in_specs=[blk128, blk128, blk16,
                _full((16, 128)), _full((16, 128)),
                _full((128, 128)), _full((128, 128)),
                _full((128, 64)), _full((128, 64))],
      out_specs=[blk128, blk64],
      out_shape=[jax.ShapeDtypeStruct((_N2, 128), F32),
                 jax.ShapeDtypeStruct((_N2, 64), F32)],
  )(s1, y1, st, wnbc, wnba, wpa, wpc, wsc2, wsa2)


def _tc3_body(s2_ref, y2_ref, st_ref, wnbc, wnba, h2_ref):
  cust = pl.program_id(0) < 5
  st = st_ref[...]
  deg = jnp.maximum(st[:, 4], 1.0)
  wnb = jnp.where(cust, wnbc[...], wnba[...])
  agg = (y2_ref[:, :64] + jnp.dot(st, wnb, preferred_element_type=F32)) / deg[:, None]
  h2 = s2_ref[...] + agg
  h2_ref[...] = jnp.concatenate([h2, jnp.zeros_like(h2)], axis=1)


def _tc3(s2, y2, st, wnbc, wnba):
  blk128 = pl.BlockSpec((1000, 128), lambda i: (i, 0))
  blk64 = pl.BlockSpec((1000, 64), lambda i: (i, 0))
  blk16 = pl.BlockSpec((1000, 16), lambda i: (i, 0))
  return pl.pallas_call(
      _tc3_body,
      grid=(10,),
      in_specs=[blk64, blk128, blk16, _full((16, 64)), _full((16, 64))],
      out_specs=blk128,
      out_shape=jax.ShapeDtypeStruct((_N2, 128), F32),
  )(s2, y2, st, wnbc, wnba)


def _tc4_body(ga_ref, gb_ref, out_ref):
  a = ga_ref[...]
  b = gb_ref[...]
  dot = jnp.sum(a * b, axis=1)
  na = jnp.sqrt(jnp.sum(a * a, axis=1))
  nb = jnp.sqrt(jnp.sum(b * b, axis=1))
  out_ref[...] = dot / (na * nb + 1e-8)


def _tc4(ga, gb):
  blk = pl.BlockSpec((6144, 128), lambda i: (i, 0))
  return pl.pallas_call(
      _tc4_body,
      grid=(10,),
      in_specs=[blk, blk],
      out_specs=pl.BlockSpec((6144,), lambda i: (i,)),
      out_shape=jax.ShapeDtypeStruct((_QPAD,), F32),
  )(ga, gb)


def _pad16(w4):
  return jnp.concatenate([w4, jnp.zeros((12, w4.shape[1]), F32)], axis=0)


def kernel(h_customer, h_article, e_feat, src_idx, dst_idx,
           pos_src, pos_dst, neg_src, neg_dst,
           W_ue, W_ie, W1a_n, W1a_s, W1c_n, W1c_s,
           W2a_n, W2a_s, W2c_n, W2c_s):
  src_idx = src_idx.astype(I32)
  dst_idx = dst_idx.astype(I32)
  npad = _EPAD - _E
  padv = jnp.arange(npad, dtype=I32) % _N
  dumpv = _N + (jnp.arange(npad, dtype=I32) % _DUMP)
  src_g = jnp.concatenate([src_idx, padv])
  dst_g = jnp.concatenate([dst_idx, padv])
  gidx = jnp.stack([dst_g + _N, src_g]).reshape(2, _EPAD // _C, _C)
  sidx = jnp.stack([jnp.concatenate([src_idx, dumpv]),
                    jnp.concatenate([dst_idx, dumpv])
                    ]).reshape(2, _EPAD // _C, _C)
  gs = jnp.stack([gidx, sidx], axis=2)
  e1 = jnp.concatenate(
      [e_feat, jnp.ones((_E, 1), F32), jnp.zeros((_E, 11), F32)], axis=1)
  e1 = jnp.concatenate([e1, jnp.zeros((npad, 16), F32)], axis=0)

  qpadv = jnp.arange(_QPAD - _Q, dtype=I32) % _N
  qa = jnp.concatenate([pos_src.astype(I32), neg_src.astype(I32), qpadv])
  qb = jnp.concatenate([pos_dst.astype(I32), neg_dst.astype(I32), qpadv]) + _N

  H = jnp.concatenate([h_customer, h_article], axis=0)

  sp = _stats_call(e1, src_g, dst_g)[:, :, :, :16]
  stats = jnp.concatenate([sp[0, 0] + sp[1, 0], sp[0, 1] + sp[1, 1]])

  x1, s1 = _tc1(H, W_ue, W_ie, W1a_n[:128], W1c_n[:128], W1c_s, W1a_s)
  y1 = _spmm_call(x1, gs)[:, :_N, :].reshape(_N2, 128)
  wpa = jnp.concatenate([W2a_n[:128], jnp.zeros((128, 64), F32)], axis=1)
  wpc = jnp.concatenate([W2c_n[:128], jnp.zeros((128, 64), F32)], axis=1)
  x2, s2 = _tc2(s1, y1, stats, _pad16(W1c_n[128:]), _pad16(W1a_n[128:]),
                wpa, wpc, W2c_s, W2a_s)
  y2 = _spmm_call(x2, gs)[:, :_N, :].reshape(_N2, 128)
  h2 = _tc3(s2, y2, stats, _pad16(W2c_n[128:]), _pad16(W2a_n[128:]))

  ga, gb = _pairs_call(h2, qa, qb)
  scores = _tc4(ga, gb)

  return h2[:_N, :64], h2[_N:, :64], scores[:_P], scores[_P:_Q]


# speedup vs baseline: 1.3174x; 1.0355x over previous
import functools

import jax
import jax.numpy as jnp
from jax import lax
from jax.experimental import pallas as pl
from jax.experimental.pallas import tpu as pltpu
from jax.experimental.pallas import tpu_sc as plsc

F32 = jnp.float32
I32 = jnp.int32

_N = 5000
_N2 = 2 * _N
_E = 320000
_EPAD = 327680
_C = 128
_EPT = _EPAD // 16
_NITER = _EPT // _C
_HEPT = _EPAD // 32
_HNITER = _HEPT // _C
_SR = 64
_SNIT = _HEPT // _SR
_DUMP = 16
_NACC = _N + _DUMP
_P = 10000
_NEG = 50000
_Q = _P + _NEG
_QPAD = 61440
_QPW = _QPAD // 32

_mesh = plsc.VectorSubcoreMesh(core_axis_name="c", subcore_axis_name="s",
                               num_cores=2, num_subcores=16)


def _zero_fill(z_v, nrow, ncol):
  def row(r, carry):
    for j in range(ncol // 16):
      z_v[r, pl.ds(j * 16, 16)] = jnp.zeros((16,), F32)
    return carry
  lax.fori_loop(0, nrow, row, 0)


def _stats_body(e1_hbm, isrc_hbm, idst_hbm, out_hbm,
                accs_sp, accd_sp, u0, u1, e16_v, xs0, xs1, xd0, xd1,
                ss0, ss1, sd0, sd1):
  c = lax.axis_index("c")
  s = lax.axis_index("s")
  upd = (u0, u1)
  ixs = (xs0, xs1)
  ixd = (xd0, xd1)
  ssem = (ss0, ss1)
  dsem = (sd0, sd1)
  _zero_fill(u0, _C, 128)
  _zero_fill(u1, _C, 128)
  for o in (0, 128, 192):
    zoff = jnp.minimum(s * 320 + o, _N - _C)
    pltpu.sync_copy(u0, accs_sp.at[pl.ds(zoff, _C)])
    pltpu.sync_copy(u0, accd_sp.at[pl.ds(zoff, _C)])
  plsc.subcore_barrier()

  def load_expand(i, b):
    base = c * (_EPAD // 2) + s * _HEPT + i * _C
    pltpu.sync_copy(isrc_hbm.at[pl.ds(base, _C)], ixs[b])
    pltpu.sync_copy(idst_hbm.at[pl.ds(base, _C)], ixd[b])
    pltpu.sync_copy(e1_hbm.at[pl.ds(base, _C)], e16_v)

    def expand(r8, c2):
      for k in range(8):
        r = r8 * 8 + k
        upd[b][r, pl.ds(0, 16)] = e16_v[r, pl.ds(0, 16)]
      return c2

    lax.fori_loop(0, _C // 8, expand, 0)

  def s_start(b):
    pltpu.async_copy(upd[b], accs_sp.at[ixs[b]], ssem[b], add=True)
    pltpu.async_copy(upd[b], accd_sp.at[ixd[b]], dsem[b], add=True)

  def s_wait(b):
    pltpu.make_async_copy(upd[b], accs_sp.at[ixs[b]], ssem[b]).wait()
    pltpu.make_async_copy(upd[b], accd_sp.at[ixd[b]], dsem[b]).wait()

  load_expand(0, 0)
  s_start(0)
  load_expand(1, 1)
  s_start(1)

  def step(j, carry):
    for b in range(2):
      i = 2 * j + b
      s_wait(b)
      load_expand(i, b)
      s_start(b)
    return carry

  lax.fori_loop(1, _HNITER // 2, step, 0)
  s_wait(0)
  s_wait(1)
  plsc.subcore_barrier()
  ooff = jnp.minimum(s * 320, _N - 320)
  pltpu.sync_copy(accs_sp.at[pl.ds(ooff, 320)],
                  out_hbm.at[c, 0, pl.ds(ooff, 320)])
  pltpu.sync_copy(accd_sp.at[pl.ds(ooff, 320)],
                  out_hbm.at[c, 1, pl.ds(ooff, 320)])


_stats_call = functools.partial(
    pl.kernel,
    _stats_body,
    out_type=jax.ShapeDtypeStruct((2, 2, _N, 128), F32),
    mesh=_mesh,
    scratch_types=[
        pltpu.VMEM_SHARED((_N, 128), F32),
        pltpu.VMEM_SHARED((_N, 128), F32),
        pltpu.VMEM((_C, 128), F32),
        pltpu.VMEM((_C, 128), F32),
        pltpu.VMEM((_C, 16), F32),
        pltpu.VMEM((_C,), I32),
        pltpu.VMEM((_C,), I32),
        pltpu.VMEM((_C,), I32),
        pltpu.VMEM((_C,), I32),
        pltpu.SemaphoreType.DMA,
        pltpu.SemaphoreType.DMA,
        pltpu.SemaphoreType.DMA,
        pltpu.SemaphoreType.DMA,
    ],
)()


def _spmm_body(tbl_hbm, gs_hbm, out_hbm,
               acc_sp, b0, b1, b2, b3, i0, i1, i2, i3,
               g0, g1, g2, g3, t0, t1, t2, t3):
  c = lax.axis_index("c")
  s = lax.axis_index("s")
  bufs = (b0, b1, b2, b3)
  ibufs = (i0, i1, i2, i3)
  gsem = (g0, g1, g2, g3)
  ssem = (t0, t1, t2, t3)
  _zero_fill(b0, _C, 128)
  for o in (0, 128, 192):
    zoff = jnp.minimum(s * 320 + o, _NACC - _C)
    pltpu.sync_copy(b0, acc_sp.at[pl.ds(zoff, _C)])
  plsc.subcore_barrier()

  def load_idx(i, b):
    pltpu.sync_copy(gs_hbm.at[c, s * _NITER + i], ibufs[b])

  def g_start(b):
    pltpu.async_copy(tbl_hbm.at[ibufs[b].at[0]], bufs[b], gsem[b])

  def g_wait(b):
    pltpu.make_async_copy(tbl_hbm.at[ibufs[b].at[0]], bufs[b], gsem[b]).wait()

  def s_start(b):
    pltpu.async_copy(bufs[b], acc_sp.at[ibufs[b].at[1]], ssem[b], add=True)

  def s_wait(b):
    pltpu.make_async_copy(bufs[b], acc_sp.at[ibufs[b].at[1]], ssem[b]).wait()

  load_idx(0, 0)
  g_start(0)
  load_idx(1, 1)
  g_start(1)
  load_idx(2, 2)
  g_start(2)
  g_wait(0)
  s_start(0)
  load_idx(3, 3)
  g_start(3)
  g_wait(1)
  s_start(1)

  def step(j, carry):
    for b in range(4):
      i = 4 * j + b
      s_wait(b)
      load_idx(i, b)
      g_start(b)
      bp = (b + 2) % 4
      g_wait(bp)
      s_start(bp)
    return carry

  lax.fori_loop(1, _NITER // 4, step, 0)
  g_wait(2)
  s_start(2)
  g_wait(3)
  s_start(3)
  for b in range(4):
    s_wait(b)
  plsc.subcore_barrier()
  ooff = jnp.minimum(s * 320, _NACC - 320)
  pltpu.sync_copy(acc_sp.at[pl.ds(ooff, 320)], out_hbm.at[c, pl.ds(ooff, 320)])


_spmm_call = functools.partial(
    pl.kernel,
    _spmm_body,
    out_type=jax.ShapeDtypeStruct((2, _NACC, 128), F32),
    mesh=_mesh,
    scratch_types=[
        pltpu.VMEM_SHARED((_NACC, 128), F32),
        pltpu.VMEM((_C, 128), F32),
        pltpu.VMEM((_C, 128), F32),
        pltpu.VMEM((_C, 128), F32),
        pltpu.VMEM((_C, 128), F32),
        pltpu.VMEM((2, _C), I32),
        pltpu.VMEM((2, _C), I32),
        pltpu.VMEM((2, _C), I32),
        pltpu.VMEM((2, _C), I32),
        pltpu.SemaphoreType.DMA,
        pltpu.SemaphoreType.DMA,
        pltpu.SemaphoreType.DMA,
        pltpu.SemaphoreType.DMA,
        pltpu.SemaphoreType.DMA,
        pltpu.SemaphoreType.DMA,
        pltpu.SemaphoreType.DMA,
        pltpu.SemaphoreType.DMA,
    ],
)()


def _pairs_body(tbl_hbm, qa_hbm, qb_hbm, oa_hbm, ob_hbm,
                ia0, ia1, ib0, ib1, ra0, ra1, rb0, rb1,
                ga0, ga1, gb0, gb1, wa0, wa1, wb0, wb1):
  c = lax.axis_index("c")
  s = lax.axis_index("s")
  wid = c * 16 + s
  ia = (ia0, ia1)
  ib = (ib0, ib1)
  ra = (ra0, ra1)
  rb = (rb0, rb1)
  gsa = (ga0, ga1)
  gsb = (gb0, gb1)
  wsa = (wa0, wa1)
  wsb = (wb0, wb1)
  nst = _QPW // _C

  def load(i, b):
    base = wid * _QPW + i * _C
    pltpu.sync_copy(qa_hbm.at[pl.ds(base, _C)], ia[b])
    pltpu.sync_copy(qb_hbm.at[pl.ds(base, _C)], ib[b])

  def g_start(b):
    pltpu.async_copy(tbl_hbm.at[ia[b]], ra[b], gsa[b])
    pltpu.async_copy(tbl_hbm.at[ib[b]], rb[b], gsb[b])

  def g_wait(b):
    pltpu.make_async_copy(tbl_hbm.at[ia[b]], ra[b], gsa[b]).wait()
    pltpu.make_async_copy(tbl_hbm.at[ib[b]], rb[b], gsb[b]).wait()

  def w_start(i, b):
    base = wid * _QPW + i * _C
    pltpu.async_copy(ra[b], oa_hbm.at[pl.ds(base, _C)], wsa[b])
    pltpu.async_copy(rb[b], ob_hbm.at[pl.ds(base, _C)], wsb[b])

  def w_wait(i, b):
    base = wid * _QPW + i * _C
    pltpu.make_async_copy(ra[b], oa_hbm.at[pl.ds(base, _C)], wsa[b]).wait()
    pltpu.make_async_copy(rb[b], ob_hbm.at[pl.ds(base, _C)], wsb[b]).wait()

  load(0, 0)
  g_start(0)
  load(1, 1)
  g_start(1)
  for i in range(nst):
    b = i % 2
    g_wait(b)
    w_start(i, b)
    if i + 2 < nst:
      w_wait(i, b)
      load(i + 2, b)
      g_start(b)
  w_wait(nst - 2, (nst - 2) % 2)
  w_wait(nst - 1, (nst - 1) % 2)


_pairs_call = functools.partial(
    pl.kernel,
    _pairs_body,
    out_type=(jax.ShapeDtypeStruct((_QPAD, 128), F32),
              jax.ShapeDtypeStruct((_QPAD, 128), F32)),
    mesh=_mesh,
    scratch_types=[
        pltpu.VMEM((_C,), I32),
        pltpu.VMEM((_C,), I32),
        pltpu.VMEM((_C,), I32),
        pltpu.VMEM((_C,), I32),
        pltpu.VMEM((_C, 128), F32),
        pltpu.VMEM((_C, 128), F32),
        pltpu.VMEM((_C, 128), F32),
        pltpu.VMEM((_C, 128), F32),
        pltpu.SemaphoreType.DMA,
        pltpu.SemaphoreType.DMA,
        pltpu.SemaphoreType.DMA,
        pltpu.SemaphoreType.DMA,
        pltpu.SemaphoreType.DMA,
        pltpu.SemaphoreType.DMA,
        pltpu.SemaphoreType.DMA,
        pltpu.SemaphoreType.DMA,
    ],
)()


def _full(shape):
  return pl.BlockSpec(shape, lambda i: (0,) * len(shape))


def _tc1_body(h_ref, wue, wie, wna, wnc, wsc, wsa, x1_ref, s1_ref):
  cust = pl.program_id(0) < 5
  we = jnp.where(cust, wue[...], wie[...])
  hb = jnp.dot(h_ref[...], we, preferred_element_type=F32)
  wn = jnp.where(cust, wna[...], wnc[...])
  x1_ref[...] = jnp.dot(hb, wn, preferred_element_type=F32)
  ws = jnp.where(cust, wsc[...], wsa[...])
  s1_ref[...] = jnp.dot(hb, ws, preferred_element_type=F32)


def _tc1(h, wue, wie, wna, wnc, wsc, wsa):
  blk = pl.BlockSpec((1000, 128), lambda i: (i, 0))
  return pl.pallas_call(
      _tc1_body,
      grid=(10,),
      in_specs=[blk] + [_full((128, 128))] * 6,
      out_specs=[blk, blk],
      out_shape=[jax.ShapeDtypeStruct((_N2, 128), F32)] * 2,
  )(h, wue, wie, wna, wnc, wsc, wsa)


def _tc2_body(s1_ref, y1_ref, st_ref, wnbc, wnba, wpa, wpc, wsc2, wsa2,
              x2_ref, s2_ref):
  cust = pl.program_id(0) < 5
  st = st_ref[...]
  deg = jnp.maximum(st[:, 4], 1.0)
  wnb = jnp.where(cust, wnbc[...], wnba[...])
  agg = (y1_ref[...] + jnp.dot(st, wnb, preferred_element_type=F32)) / deg[:, None]
  h1 = jnp.maximum(s1_ref[...] + agg, 0.0)
  x2_ref[...] = jnp.dot(h1, jnp.where(cust, wpa[...], wpc[...]),
                        preferred_element_type=F32)
  s2_ref[...] = jnp.dot(h1, jnp.where(cust, wsc2[...], wsa2[...]),
                        preferred_element_type=F32)


def _tc2(s1, y1, st, wnbc, wnba, wpa, wpc, wsc2, wsa2):
  blk128 = pl.BlockSpec((1000, 128), lambda i: (i, 0))
  blk64 = pl.BlockSpec((1000, 64), lambda i: (i, 0))
  blk16 = pl.BlockSpec((1000, 16), lambda i: (i, 0))
  return pl.pallas_call(
      _tc2_body,
      grid=(10,),
      in_specs=[blk128, blk128, blk16,
                _full((16, 128)), _full((16, 128)),
                _full((128, 128)), _full((128, 128)),
                _full((128, 64)), _full((128, 64))],
      out_specs=[blk128, blk64],
      out_shape=[jax.ShapeDtypeStruct((_N2, 128), F32),
                 jax.ShapeDtypeStruct((_N2, 64), F32)],
  )(s1, y1, st, wnbc, wnba, wpa, wpc, wsc2, wsa2)


def _tc3_body(s2_ref, y2_ref, st_ref, wnbc, wnba, h2_ref):
  cust = pl.program_id(0) < 5
  st = st_ref[...]
  deg = jnp.maximum(st[:, 4], 1.0)
  wnb = jnp.where(cust, wnbc[...], wnba[...])
  agg = (y2_ref[:, :64] + jnp.dot(st, wnb, preferred_element_type=F32)) / deg[:, None]
  h2 = s2_ref[...] + agg
  h2_ref[...] = jnp.concatenate([h2, jnp.zeros_like(h2)], axis=1)


def _tc3(s2, y2, st, wnbc, wnba):
  blk128 = pl.BlockSpec((1000, 128), lambda i: (i, 0))
  blk64 = pl.BlockSpec((1000, 64), lambda i: (i, 0))
  blk16 = pl.BlockSpec((1000, 16), lambda i: (i, 0))
  return pl.pallas_call(
      _tc3_body,
      grid=(10,),
      in_specs=[blk64, blk128, blk16, _full((16, 64)), _full((16, 64))],
      out_specs=blk128,
      out_shape=jax.ShapeDtypeStruct((_N2, 128), F32),
  )(s2, y2, st, wnbc, wnba)


def _tc4_body(ga_ref, gb_ref, out_ref):
  a = ga_ref[...]
  b = gb_ref[...]
  dot = jnp.sum(a * b, axis=1)
  na = jnp.sqrt(jnp.sum(a * a, axis=1))
  nb = jnp.sqrt(jnp.sum(b * b, axis=1))
  out_ref[...] = dot / (na * nb + 1e-8)


def _tc4(ga, gb):
  blk = pl.BlockSpec((6144, 128), lambda i: (i, 0))
  return pl.pallas_call(
      _tc4_body,
      grid=(10,),
      in_specs=[blk, blk],
      out_specs=pl.BlockSpec((6144,), lambda i: (i,)),
      out_shape=jax.ShapeDtypeStruct((_QPAD,), F32),
  )(ga, gb)


def _pad16(w4):
  return jnp.concatenate([w4, jnp.zeros((12, w4.shape[1]), F32)], axis=0)


def kernel(h_customer, h_article, e_feat, src_idx, dst_idx,
           pos_src, pos_dst, neg_src, neg_dst,
           W_ue, W_ie, W1a_n, W1a_s, W1c_n, W1c_s,
           W2a_n, W2a_s, W2c_n, W2c_s):
  src_idx = src_idx.astype(I32)
  dst_idx = dst_idx.astype(I32)
  npad = _EPAD - _E
  padv = jnp.arange(npad, dtype=I32) % _N
  dumpv = _N + (jnp.arange(npad, dtype=I32) % _DUMP)
  src_g = jnp.concatenate([src_idx, padv])
  dst_g = jnp.concatenate([dst_idx, padv])
  gidx = jnp.stack([dst_g + _N, src_g]).reshape(2, _EPAD // _C, _C)
  sidx = jnp.stack([jnp.concatenate([src_idx, dumpv]),
                    jnp.concatenate([dst_idx, dumpv])
                    ]).reshape(2, _EPAD // _C, _C)
  gs = jnp.stack([gidx, sidx], axis=2)
  e1 = jnp.concatenate(
      [e_feat, jnp.ones((_E, 1), F32), jnp.zeros((_E, 11), F32)], axis=1)
  e1 = jnp.concatenate([e1, jnp.zeros((npad, 16), F32)], axis=0)

  qpadv = jnp.arange(_QPAD - _Q, dtype=I32) % _N
  qa = jnp.concatenate([pos_src.astype(I32), neg_src.astype(I32), qpadv])
  qb = jnp.concatenate([pos_dst.astype(I32), neg_dst.astype(I32), qpadv]) + _N

  H = jnp.concatenate([h_customer, h_article], axis=0)

  sp = _stats_call(e1, src_g, dst_g)[:, :, :, :16]
  stats = jnp.concatenate([sp[0, 0] + sp[1, 0], sp[0, 1] + sp[1, 1]])

  x1, s1 = _tc1(H, W_ue, W_ie, W1a_n[:128], W1c_n[:128], W1c_s, W1a_s)
  y1 = _spmm_call(x1, gs)[:, :_N, :].reshape(_N2, 128)
  wpa = jnp.concatenate([W2a_n[:128], jnp.zeros((128, 64), F32)], axis=1)
  wpc = jnp.concatenate([W2c_n[:128], jnp.zeros((128, 64), F32)], axis=1)
  x2, s2 = _tc2(s1, y1, stats, _pad16(W1c_n[128:]), _pad16(W1a_n[128:]),
                wpa, wpc, W2c_s, W2a_s)
  y2 = _spmm_call(x2, gs)[:, :_N, :].reshape(_N2, 128)
  h2 = _tc3(s2, y2, stats, _pad16(W2c_n[128:]), _pad16(W2a_n[128:]))

  ga, gb = _pairs_call(h2, qa, qb)
  scores = _tc4(ga, gb)

  return h2[:_N, :64], h2[_N:, :64], scores[:_P], scores[_P:_Q]
